# baseline (speedup 1.0000x reference)
import functools
import math

import jax
import jax.numpy as jnp
from jax import lax
from jax.experimental import pallas as pl
from jax.experimental.pallas import tpu as pltpu
from jax.experimental.pallas import tpu_sc as plsc

VOCAB = 1000000
D = 64
BATCH = 4096
HIST = 200
N = BATCH * HIST
SCALE = math.sqrt(D)

NC = 2
NS = 16
NW = NC * NS
BPW = N // NW
K = 512
CHUNKS = BPW // K
LANES = 16


def _emb_body(x_hbm, lut_hbm, out_hbm, idx_v, rows_v, gsem):
    wid = lax.axis_index("s") * NC + lax.axis_index("c")
    base = wid * BPW
    pltpu.sync_copy(x_hbm.at[pl.ds(base, BPW)], idx_v)

    def chunk_body(ci, carry):
        off = ci * K
        pltpu.async_copy(lut_hbm.at[idx_v.at[pl.ds(off, K)]], rows_v, gsem).wait()

        def row_body(r, c2):
            for c in range(D // LANES):
                sl = (r, pl.ds(c * LANES, LANES))
                rows_v[sl] = rows_v[sl] * SCALE
            return c2

        lax.fori_loop(0, K, row_body, 0)
        pltpu.sync_copy(rows_v, out_hbm.at[pl.ds(base + off, K)])
        return carry

    lax.fori_loop(0, CHUNKS, chunk_body, 0)


@jax.jit
def _emb(x_flat, lut):
    mesh = plsc.VectorSubcoreMesh(core_axis_name="c", subcore_axis_name="s")
    return pl.kernel(
        _emb_body,
        out_type=jax.ShapeDtypeStruct((N, D), jnp.float32),
        mesh=mesh,
        scratch_types=[
            pltpu.VMEM((BPW,), jnp.int32),
            pltpu.VMEM((K, D), jnp.float32),
            pltpu.SemaphoreType.DMA,
        ],
        compiler_params=pltpu.CompilerParams(use_tc_tiling_on_sc=False),
    )(x_flat, lut)


def kernel(x, lut):
    x_flat = x.reshape(-1).astype(jnp.int32)
    out = _emb(x_flat, lut)
    return out.reshape(BATCH, HIST, D)


# speedup vs baseline: 1.1197x; 1.1197x over previous
import math

import jax
import jax.numpy as jnp
from jax import lax
from jax.experimental import pallas as pl
from jax.experimental.pallas import tpu as pltpu
from jax.experimental.pallas import tpu_sc as plsc

VOCAB = 1000000
D = 64
BATCH = 4096
HIST = 200
N = BATCH * HIST
SCALE = math.sqrt(D)

NC = 2
NS = 16
NW = NC * NS
BPW = N // NW
K = 320
CHUNKS = BPW // K
NBUF = 4
LANES = 16
RUNROLL = 4


def _emb_body(x_hbm, lut_hbm, out_hbm, idx_v, rows_v, *sems):
    gsem = sems[:NBUF]
    osem = sems[NBUF:]
    wid = lax.axis_index("s") * NC + lax.axis_index("c")
    base = wid * BPW
    pltpu.sync_copy(x_hbm.at[pl.ds(base, BPW)], idx_v)

    def g_copy(chunk, b):
        return pltpu.make_async_copy(
            lut_hbm.at[idx_v.at[pl.ds(chunk * K, K)]], rows_v.at[b], gsem[b])

    def o_copy(chunk, b):
        return pltpu.make_async_copy(
            rows_v.at[b], out_hbm.at[pl.ds(base + chunk * K, K)], osem[b])

    for b in range(NBUF - 1):
        g_copy(b, b).start()

    def outer(it, carry):
        ci = it * NBUF
        for b in range(NBUF):
            chunk = ci + b
            g_copy(chunk, b).wait()

            def row_step(r):
                for rr in range(RUNROLL):
                    for c in range(D // LANES):
                        sl = (r + rr, pl.ds(c * LANES, LANES))
                        rows_v.at[b][sl] = rows_v.at[b][sl] * SCALE

            plsc.parallel_loop(0, K, RUNROLL)(row_step)

            o_copy(chunk, b).start()

            nb = (b + NBUF - 1) % NBUF
            g = chunk + NBUF - 1
            if b == 0:
                @pl.when(it > 0)
                def _():
                    o_copy(chunk - 1, nb).wait()
            else:
                o_copy(chunk - 1, nb).wait()

            @pl.when(g < CHUNKS)
            def _():
                g_copy(g, nb).start()
        return carry

    lax.fori_loop(0, CHUNKS // NBUF, outer, 0)

    o_copy(CHUNKS - 1, (CHUNKS - 1) % NBUF).wait()


@jax.jit
def _emb(x_flat, lut):
    mesh = plsc.VectorSubcoreMesh(core_axis_name="c", subcore_axis_name="s")
    return pl.kernel(
        _emb_body,
        out_type=jax.ShapeDtypeStruct((N, D), jnp.float32),
        mesh=mesh,
        scratch_types=[
            pltpu.VMEM((BPW,), jnp.int32),
            pltpu.VMEM((NBUF, K, D), jnp.float32),
        ] + [pltpu.SemaphoreType.DMA] * (2 * NBUF),
        compiler_params=pltpu.CompilerParams(use_tc_tiling_on_sc=False),
    )(x_flat, lut)


def kernel(x, lut):
    x_flat = x.reshape(-1).astype(jnp.int32)
    out = _emb(x_flat, lut)
    return out.reshape(BATCH, HIST, D)
